# baseline (speedup 1.0000x reference)
import functools

import jax
import jax.numpy as jnp
from jax import lax
from jax.experimental import pallas as pl
from jax.experimental.pallas import tpu as pltpu
from jax.experimental.pallas import tpu_sc as plsc

_V = 1000000
_D = 32
_B0 = 16384
_H = 50
_TAIL0 = (_V // 128) * 128
_NTAIL = _V - _TAIL0
_NBT = _B0 // 128
_SPAN = 128 * _H


def _gather_call(NC):
    mesh = plsc.VectorSubcoreMesh(core_axis_name="c", subcore_axis_name="s")
    NW = NC * 16
    bt_per_w = _NBT // NW
    nchunks = bt_per_w * _H

    @functools.partial(
        pl.kernel,
        mesh=mesh,
        out_type=jax.ShapeDtypeStruct((_H, _D // 8, _B0 // 128, 8, 128),
                                      jnp.float32),
        scratch_types=[
            pltpu.VMEM((_SPAN,), jnp.int32),
            pltpu.VMEM((128,), jnp.int32),
            pltpu.VMEM((128,), jnp.int32),
            pltpu.VMEM((128, _D), jnp.float32),
            pltpu.VMEM((128, _D), jnp.float32),
            pltpu.VMEM((_D, 128), jnp.float32),
            pltpu.VMEM((_D, 128), jnp.float32),
            pltpu.VMEM((_NTAIL, _D), jnp.float32),
            pltpu.SemaphoreType.DMA,
            pltpu.SemaphoreType.DMA,
            pltpu.SemaphoreType.DMA,
            pltpu.SemaphoreType.DMA,
        ],
        compiler_params=pltpu.CompilerParams(
            use_tc_tiling_on_sc=False, needs_layout_passes=False),
    )
    def k(ids_hbm, tab_hbm, tail_hbm, out_hbm, span, ib0, ib1, r0, r1,
          w0, w1, tailv, sg0, sg1, so0, so1):
        wid = lax.axis_index("s") * NC + lax.axis_index("c")
        ibuf = [ib0, ib1]
        rows = [r0, r1]
        wbuf = [w0, w1]
        sg = [sg0, sg1]
        so = [so0, so1]
        iota = lax.iota(jnp.int32, 16)
        iota50 = iota * _H
        rowvecs = [iota + seg * 16 for seg in range(8)]

        pltpu.sync_copy(tail_hbm, tailv)

        def extract(c, p):
            bt_local = c // _H
            h = c - bt_local * _H
            bt_g = wid * bt_per_w + bt_local

            if isinstance(c, int):
                if h == 0:
                    pltpu.sync_copy(
                        ids_hbm.at[pl.ds(bt_g * _SPAN, _SPAN)], span)
            else:
                @pl.when(h == 0)
                def _():
                    pltpu.sync_copy(
                        ids_hbm.at[pl.ds(bt_g * _SPAN, _SPAN)], span)

            for seg in range(8):
                offs = iota50 + (seg * 16 * _H + h)
                v = plsc.load_gather(span, [offs])
                ibuf[p][pl.ds(seg * 16, 16)] = v
            return pltpu.async_copy(tab_hbm.at[ibuf[p]], rows[p], sg[p])

        def patch(p):
            for seg in range(8):
                idxv = ibuf[p][pl.ds(seg * 16, 16)]
                tmask = idxv >= _TAIL0
                cnt = jnp.max(jnp.where(tmask, 1, 0))

                @pl.when(cnt > 0)
                def _():
                    sel = jnp.where(tmask, idxv - _TAIL0, 0)
                    for j in range(_D):
                        g = plsc.load_gather(
                            tailv, [sel, jnp.full((16,), j, jnp.int32)],
                            mask=tmask)
                        plsc.store_scatter(
                            rows[p], [rowvecs[seg],
                                      jnp.full((16,), j, jnp.int32)],
                            g, mask=tmask)

        def transpose(p):
            for j in range(_D):
                for seg in range(8):
                    v = plsc.load_gather(
                        rows[p], [rowvecs[seg],
                                  jnp.full((16,), j, jnp.int32)])
                    wbuf[p][j, pl.ds(seg * 16, 16)] = v

        def fire_writes(c, p):
            bt_local = c // _H
            h = c - bt_local * _H
            bt_g = wid * bt_per_w + bt_local
            for jb in range(_D // 8):
                pltpu.async_copy(wbuf[p].at[pl.ds(jb * 8, 8)],
                                 out_hbm.at[h, jb, bt_g], so[p])

        def drain(p):
            for _ in range(_D // 8):
                pltpu.make_async_copy(out_hbm.at[0, 0, 0],
                                      wbuf[p].at[pl.ds(0, 8)], so[p]).wait()

        g0 = extract(0, 0)
        del g0

        def body(g, carry):
            cA = 2 * g
            cB = 2 * g + 1
            extract(cB, 1)
            pltpu.make_async_copy(tab_hbm.at[ibuf[0]], rows[0], sg[0]).wait()
            patch(0)

            @pl.when(g > 0)
            def _():
                drain(0)

            transpose(0)
            fire_writes(cA, 0)

            @pl.when(g < nchunks // 2 - 1)
            def _():
                extract(cB + 1, 0)

            pltpu.make_async_copy(tab_hbm.at[ibuf[1]], rows[1], sg[1]).wait()
            patch(1)

            @pl.when(g > 0)
            def _():
                drain(1)

            transpose(1)
            fire_writes(cB, 1)
            return carry

        lax.fori_loop(0, nchunks // 2, body, 0)
        drain(0)
        drain(1)

    return k


def kernel(batch_ids, table):
    ids = batch_ids.reshape(_B0 * _H).astype(jnp.int32)
    tail = lax.slice(table, (_TAIL0, 0), (_V, _D))
    info = plsc.get_sparse_core_info()
    outk = _gather_call(info.num_cores)(ids, table, tail)
    return outk.transpose(2, 4, 0, 1, 3).reshape(_B0, _H, _D)


# speedup vs baseline: 1.0602x; 1.0602x over previous
import functools

import jax
import jax.numpy as jnp
from jax import lax
from jax.experimental import pallas as pl
from jax.experimental.pallas import tpu as pltpu
from jax.experimental.pallas import tpu_sc as plsc

_V = 1000000
_D = 32
_B0 = 16384
_H = 50
_NFULL = _V // 128
_TAIL0 = _NFULL * 128
_NTAIL = _V - _TAIL0
_SPAN = 512 * _H


def _transpose_call(NC):
    mesh = plsc.VectorSubcoreMesh(core_axis_name="c", subcore_axis_name="s")
    NW = NC * 16
    per_w = _NFULL // NW
    rem = _NFULL - per_w * NW
    nsu = per_w // 2
    npair = nsu // 2

    @functools.partial(
        pl.kernel,
        mesh=mesh,
        out_type=jax.ShapeDtypeStruct((_V * _D,), jnp.float32),
        scratch_types=[
            pltpu.VMEM((_D, 256), jnp.float32),
            pltpu.VMEM((_D, 256), jnp.float32),
            pltpu.VMEM((8192,), jnp.float32),
            pltpu.VMEM((8192,), jnp.float32),
            pltpu.SemaphoreType.DMA,
            pltpu.SemaphoreType.DMA,
            pltpu.SemaphoreType.DMA,
            pltpu.SemaphoreType.DMA,
        ],
        compiler_params=pltpu.CompilerParams(
            use_tc_tiling_on_sc=True, needs_layout_passes=False),
    )
    def k(tabT_hbm, out_hbm, vi0, vi1, vo0, vo1, si0, si1, so0, so1):
        wid = lax.axis_index("s") * NC + lax.axis_index("c")
        base = wid * per_w + jnp.minimum(wid, rem)
        vin = [vi0, vi1]
        vout = [vo0, vo1]
        si = [si0, si1]
        so = [so0, so1]
        iota = lax.iota(jnp.int32, 16)
        posb = [(m * 16 + iota) * _D for m in range(8)]

        def fire_in(su, p):
            off = pl.multiple_of((base + su * 2) * 128, 128)
            return pltpu.async_copy(
                tabT_hbm.at[:, pl.ds(off, 256)], vin[p], si[p])

        def wait_in(p):
            pltpu.make_async_copy(
                tabT_hbm.at[:, pl.ds(0, 256)], vin[p], si[p]).wait()

        def drain_out(p):
            pltpu.make_async_copy(
                out_hbm.at[pl.ds(0, 8192)], vout[p], so[p]).wait()

        def transpose(p):
            for u in range(2):
                for j in range(_D):
                    for seg in range(8):
                        v = vin[p][j, pl.ds(u * 128 + seg * 16, 16)]
                        plsc.store_scatter(
                            vout[p], [posb[seg] + (u * 4096 + j)], v)

        def fire_out(su, p):
            off = pl.multiple_of((base + su * 2) * 4096, 4096)
            return pltpu.async_copy(
                vout[p], out_hbm.at[pl.ds(off, 8192)], so[p])

        fire_in(0, 0)

        def body(g, carry):
            suA = 2 * g
            suB = 2 * g + 1
            fire_in(suB, 1)
            wait_in(0)

            @pl.when(g > 0)
            def _():
                drain_out(0)

            transpose(0)
            fire_out(suA, 0)

            @pl.when(g < npair - 1)
            def _():
                fire_in(suB + 1, 0)

            wait_in(1)

            @pl.when(g > 0)
            def _():
                drain_out(1)

            transpose(1)
            fire_out(suB, 1)
            return carry

        lax.fori_loop(0, npair, body, 0)
        drain_out(0)
        drain_out(1)

        @pl.when(wid < rem)
        def _():
            it = base + per_w
            pltpu.sync_copy(tabT_hbm.at[:, pl.ds(it * 128, 128)],
                            vin[0].at[:, pl.ds(0, 128)])
            for j in range(_D):
                for seg in range(8):
                    v = vin[0][j, pl.ds(seg * 16, 16)]
                    plsc.store_scatter(vout[0], [posb[seg] + j], v)
            pltpu.sync_copy(vout[0].at[pl.ds(0, 4096)],
                            out_hbm.at[pl.ds(it * 4096, 4096)])

    return k


def _gather_call(NC):
    mesh = plsc.VectorSubcoreMesh(core_axis_name="c", subcore_axis_name="s")
    NW = NC * 16

    @functools.partial(
        pl.kernel,
        mesh=mesh,
        out_type=jax.ShapeDtypeStruct((_H, _D // 8, _B0 // 128, 8, 128),
                                      jnp.float32),
        scratch_types=[
            pltpu.VMEM((_SPAN,), jnp.int32),
            pltpu.VMEM((512,), jnp.int32),
            pltpu.VMEM((512,), jnp.int32),
            pltpu.VMEM((512, _D), jnp.float32),
            pltpu.VMEM((512, _D), jnp.float32),
            pltpu.VMEM((_D, 512), jnp.float32),
            pltpu.VMEM((_D, 512), jnp.float32),
            pltpu.VMEM((_NTAIL, _D), jnp.float32),
            pltpu.VMEM((16,), jnp.int32),
            pltpu.VMEM((16,), jnp.int32),
            pltpu.SemaphoreType.DMA,
            pltpu.SemaphoreType.DMA,
            pltpu.SemaphoreType.DMA,
            pltpu.SemaphoreType.DMA,
        ],
        compiler_params=pltpu.CompilerParams(
            use_tc_tiling_on_sc=False, needs_layout_passes=False),
    )
    def k(ids_hbm, tab_hbm, tail_hbm, out_hbm, span, ib0, ib1, r0, r1,
          w0, w1, tailv, m0, m1, sg0, sg1, so0, so1):
        wid = lax.axis_index("s") * NC + lax.axis_index("c")
        ibuf = [ib0, ib1]
        rows = [r0, r1]
        wbig = [w0, w1]
        mbuf = [m0, m1]
        sg = [sg0, sg1]
        so = [so0, so1]
        iota = lax.iota(jnp.int32, 16)
        iota50 = iota * _H
        jvecs = [m * 16 + iota for m in range(2)]

        pltpu.sync_copy(tail_hbm, tailv)
        pltpu.sync_copy(ids_hbm.at[pl.ds(wid * _SPAN, _SPAN)], span)

        def extract(h, p):
            mx = jnp.zeros((16,), jnp.int32)
            for seg in range(32):
                offs = iota50 + (seg * 16 * _H + h)
                v = plsc.load_gather(span, [offs])
                ibuf[p][pl.ds(seg * 16, 16)] = v
                mx = jnp.maximum(mx, v)
            mbuf[p][pl.ds(0, 16)] = mx
            return pltpu.async_copy(tab_hbm.at[ibuf[p]], rows[p], sg[p])

        def wait_g(p):
            pltpu.make_async_copy(tab_hbm.at[ibuf[p]], rows[p], sg[p]).wait()

        def patch(p):
            mx = jnp.max(mbuf[p][pl.ds(0, 16)])

            @pl.when(mx >= _TAIL0)
            def _():
                def pbody(s, carry):
                    idxv = ibuf[p][pl.ds(s * 16, 16)]
                    tmask = idxv >= _TAIL0
                    sel = jnp.where(tmask, idxv - _TAIL0, 0)
                    rowv = iota + s * 16
                    for j in range(_D):
                        jv = jnp.full((16,), j, jnp.int32)
                        gv = plsc.load_gather(tailv, [sel, jv], mask=tmask)
                        plsc.store_scatter(rows[p], [rowv, jv], gv,
                                           mask=tmask)
                    return carry

                lax.fori_loop(0, 32, pbody, 0)

        def transpose(p):
            def tbody(g, carry):
                b0 = g * 8
                for kk in range(16):
                    b = b0 + kk // 2
                    m = kk % 2
                    v = rows[p][b, pl.ds(m * 16, 16)]
                    bvec = b + jnp.zeros((16,), jnp.int32)
                    plsc.store_scatter(wbig[p], [jvecs[m], bvec], v)
                return carry

            lax.fori_loop(0, 64, tbody, 0)

        def fire_writes(h, p):
            for jb in range(_D // 8):
                for btl in range(4):
                    pltpu.async_copy(
                        wbig[p].at[pl.ds(jb * 8, 8), pl.ds(btl * 128, 128)],
                        out_hbm.at[h, jb, wid * 4 + btl], so[p])

        def drain(p):
            pltpu.make_async_copy(
                tab_hbm.at[pl.ds(0, 512)], rows[p], so[p]).wait()

        extract(0, 0)

        def body(g, carry):
            hA = 2 * g
            hB = 2 * g + 1
            extract(hB, 1)
            wait_g(0)
            patch(0)

            @pl.when(g > 0)
            def _():
                drain(0)

            transpose(0)
            fire_writes(hA, 0)

            @pl.when(g < _H // 2 - 1)
            def _():
                extract(hB + 1, 0)

            wait_g(1)
            patch(1)

            @pl.when(g > 0)
            def _():
                drain(1)

            transpose(1)
            fire_writes(hB, 1)
            return carry

        lax.fori_loop(0, _H // 2, body, 0)
        drain(0)
        drain(1)

    return k


def kernel(batch_ids, table):
    ids = batch_ids.reshape(_B0 * _H).astype(jnp.int32)
    tail = lax.slice(table, (_TAIL0, 0), (_V, _D))
    info = plsc.get_sparse_core_info()
    NC = info.num_cores
    tablin = _transpose_call(NC)(table.T)
    tab2 = tablin.reshape(_V, _D)
    outk = _gather_call(NC)(ids, tab2, tail)
    return outk.transpose(2, 4, 0, 1, 3).reshape(_B0, _H, _D)


# speedup vs baseline: 1.2502x; 1.1792x over previous
import functools

import jax
import jax.numpy as jnp
from jax import lax
from jax.experimental import pallas as pl
from jax.experimental.pallas import tpu as pltpu
from jax.experimental.pallas import tpu_sc as plsc


def _gather_call(B0, H, D, bs_per_w, bchunk, nchunks, NC):
    mesh = plsc.VectorSubcoreMesh(core_axis_name="c", subcore_axis_name="s")
    chunk = bchunk * H

    @functools.partial(
        pl.kernel,
        mesh=mesh,
        out_type=jax.ShapeDtypeStruct((B0, H, D), jnp.float32),
        scratch_types=[
            pltpu.VMEM((chunk,), jnp.int32),
            pltpu.VMEM((chunk,), jnp.int32),
            pltpu.VMEM((chunk, D), jnp.float32),
            pltpu.VMEM((chunk, D), jnp.float32),
            pltpu.SemaphoreType.DMA,
            pltpu.SemaphoreType.DMA,
            pltpu.SemaphoreType.DMA,
            pltpu.SemaphoreType.DMA,
        ],
        compiler_params=pltpu.CompilerParams(use_tc_tiling_on_sc=False),
    )
    def k(idx_hbm, table_hbm, out_hbm, i0, i1, r0, r1, sg0, sg1, so0, so1):
        idx_v = [i0, i1]
        rows_v = [r0, r1]
        sg = [sg0, sg1]
        so = [so0, so1]
        wid = lax.axis_index("s") * NC + lax.axis_index("c")
        b_base = wid * bs_per_w

        def issue_outs(b, c):
            b0 = b_base + c * bchunk

            def body(i, carry):
                pltpu.async_copy(
                    rows_v[b].at[pl.ds(i * H, H)], out_hbm.at[b0 + i], so[b])
                return carry

            lax.fori_loop(0, bchunk, body, 0)
            return pltpu.make_async_copy(
                table_hbm.at[pl.ds(0, chunk)], rows_v[b], so[b])

        gathers = [None, None]
        outs = [None, None]
        pltpu.sync_copy(idx_hbm.at[pl.ds(b_base * H, chunk)], idx_v[0])
        gathers[0] = pltpu.async_copy(table_hbm.at[idx_v[0]], rows_v[0], sg[0])
        for c in range(nchunks):
            b = c % 2
            nb = (c + 1) % 2
            if c + 1 < nchunks:
                off = (b_base + (c + 1) * bchunk) * H
                pltpu.sync_copy(idx_hbm.at[pl.ds(off, chunk)], idx_v[nb])
                if c >= 1:
                    outs[nb].wait()
                gathers[nb] = pltpu.async_copy(
                    table_hbm.at[idx_v[nb]], rows_v[nb], sg[nb])
            gathers[b].wait()
            outs[b] = issue_outs(b, c)
        outs[0].wait()
        outs[1].wait()

    return k


def kernel(batch_ids, table):
    B0, H = batch_ids.shape
    V, D = table.shape
    flat = batch_ids.reshape(B0 * H).astype(jnp.int32)

    info = plsc.get_sparse_core_info()
    NC, NS = info.num_cores, info.num_subcores
    NW = NC * NS
    bs_per_w = B0 // NW
    bchunk = 32
    nchunks = bs_per_w // bchunk

    return _gather_call(B0, H, D, bs_per_w, bchunk, nchunks, NC)(flat, table)
